# Initial kernel scaffold; baseline (speedup 1.0000x reference)
import jax
import jax.numpy as jnp
from jax.experimental import pallas as pl


def kernel(x, path_matrix, sibling_mask):
    raise NotImplementedError("write your pallas kernel here")


# speedup vs baseline: 3.2500x; 3.2500x over previous
import functools

import jax
import jax.numpy as jnp
from jax import lax
from jax.experimental import pallas as pl
from jax.experimental.pallas import tpu as pltpu
from jax.experimental.pallas import tpu_sc as plsc

_BRANCH = 4
_DEPTH = 5
_N_NODES = 1365
_N_TOKENS = 16384
_LN2 = 0.6931471805599453

_LOG_C = (1.0000036599605875, -0.49993654601618936, 0.33263639639207315,
          -0.25289669739154647, 0.22019340789004246, -0.15008180119238576)

_LEVEL_START = (0, 1, 5, 21, 85, 341, 1365)

_INFO = plsc.get_sparse_core_info()
_NC, _NS = _INFO.num_cores, _INFO.num_subcores
_NW = _NC * _NS
_ROWS_PER_W = _N_TOKENS // _NW
_CHUNK = 8
_CHUNKS_PER_W = _ROWS_PER_W // _CHUNK
_CHUNK_WORDS = _CHUNK * _N_NODES

_ITERS = tuple(
    (p0, min(16, _LEVEL_START[lvl] - p0))
    for lvl in range(1, _DEPTH + 1)
    for p0 in range(_LEVEL_START[lvl - 1], _LEVEL_START[lvl], 16)
)


def _log_1_4(s):
    ix = lax.bitcast_convert_type(s, jnp.int32)
    e = (ix - jnp.int32(0x3F330000)) >> 23
    m = lax.bitcast_convert_type(ix - (e << 23), jnp.float32)
    z = m - jnp.float32(1.0)
    q = jnp.float32(_LOG_C[-1])
    for c in _LOG_C[-2::-1]:
        q = q * z + jnp.float32(c)
    return z * q + e.astype(jnp.float32) * jnp.float32(_LN2)


def _body(x_hbm, out_hbm, xin, outb):
    wid = lax.axis_index("s") * _NC + lax.axis_index("c")
    iota = lax.iota(jnp.int32, 16)
    iota4 = iota * 4
    zeros16 = jnp.zeros((16,), jnp.float32)

    def chunk_body(ck, carry):
        base = (wid * _CHUNKS_PER_W + ck) * _CHUNK_WORDS
        pltpu.sync_copy(x_hbm.at[pl.ds(base, _CHUNK_WORDS)], xin)

        def row_body(i, rcarry):
            roff = i * _N_NODES
            outb[pl.ds(roff, 16)] = zeros16
            for p0, nv in _ITERS:
                idx0 = iota4 + (roff + 4 * p0 + 1)
                c0 = plsc.load_gather(xin, [idx0])
                c1 = plsc.load_gather(xin, [idx0 + 1])
                c2 = plsc.load_gather(xin, [idx0 + 2])
                c3 = plsc.load_gather(xin, [idx0 + 3])
                mx = jnp.maximum(jnp.maximum(c0, c1), jnp.maximum(c2, c3))
                s = ((jnp.exp(c0 - mx) + jnp.exp(c1 - mx))
                     + (jnp.exp(c2 - mx) + jnp.exp(c3 - mx)))
                lse = _log_1_4(s) + mx
                cp = outb[pl.ds(roff + p0, 16)]
                acc = cp - lse
                mask = None if nv == 16 else (iota < nv)
                plsc.store_scatter(outb, [idx0], c0 + acc, mask=mask)
                plsc.store_scatter(outb, [idx0 + 1], c1 + acc, mask=mask)
                plsc.store_scatter(outb, [idx0 + 2], c2 + acc, mask=mask)
                plsc.store_scatter(outb, [idx0 + 3], c3 + acc, mask=mask)
            return rcarry

        lax.fori_loop(0, _CHUNK, row_body, 0)
        pltpu.sync_copy(outb, out_hbm.at[pl.ds(base, _CHUNK_WORDS)])
        return carry

    lax.fori_loop(0, _CHUNKS_PER_W, chunk_body, 0)


@jax.jit
def _run(xf):
    return pl.kernel(
        _body,
        out_type=jax.ShapeDtypeStruct((_N_TOKENS * _N_NODES,), jnp.float32),
        mesh=plsc.VectorSubcoreMesh(core_axis_name="c", subcore_axis_name="s"),
        scratch_types=[
            pltpu.VMEM((_CHUNK_WORDS,), jnp.float32),
            pltpu.VMEM((_CHUNK_WORDS,), jnp.float32),
        ],
        compiler_params=pltpu.CompilerParams(needs_layout_passes=False),
    )(xf)


def kernel(x, path_matrix, sibling_mask):
    del path_matrix, sibling_mask
    return _run(x.reshape(-1)).reshape(_N_TOKENS, _N_NODES)
